# baseline (speedup 1.0000x reference)
import functools

import jax
import jax.numpy as jnp
from jax import lax
from jax.experimental import pallas as pl
from jax.experimental.pallas import tpu as pltpu
from jax.experimental.pallas import tpu_sc as plsc

N = 10000
E = 320000
H = 128
G = 64
EPS = 1e-5

NC = 2
NS = 16
NW = NC * NS
CH = 256
EPW = ((E // NW + 2 * CH - 1) // (2 * CH)) * (2 * CH)
NCHUNK = EPW // CH
HALF = NCHUNK // 2
NITER_H = HALF // 2
E_PAD = EPW * NW
NPAD = ((N + NS * 8 - 1) // (NS * 8)) * (NS * 8)
ZROWS = NPAD // NS
OROWS = (N // NS) // 8 * 8
OREM = N - OROWS * NS
NP8 = N + 8


def _mesh():
    return plsc.VectorSubcoreMesh(core_axis_name="c", subcore_axis_name="s",
                                  num_cores=NC, num_subcores=NS)


def _sc_deg(dst_pad, ones128, zeros128):
    @functools.partial(
        pl.kernel,
        mesh=_mesh(),
        out_type=jax.ShapeDtypeStruct((NC, N, H), jnp.float32),
        scratch_types=[
            pltpu.VMEM((CH,), jnp.int32),
            pltpu.VMEM((CH, H), jnp.float32),
            pltpu.VMEM_SHARED((NPAD, H), jnp.float32),
            pltpu.SemaphoreType.DMA,
        ],
    )
    def k(dst_hbm, ones_hbm, zeros_hbm, out_hbm, didx, ones_v, shared, sem):
        c = lax.axis_index("c")
        s = lax.axis_index("s")
        wid = c * NS + s
        pltpu.sync_copy(zeros_hbm.at[pl.ds(s * ZROWS, ZROWS)],
                        shared.at[pl.ds(s * ZROWS, ZROWS)])
        pltpu.sync_copy(ones_hbm, ones_v)
        base = pl.multiple_of(wid * EPW, CH)
        plsc.subcore_barrier()

        def body(j, carry):
            st = pl.multiple_of(base + j * CH, CH)
            pltpu.sync_copy(dst_hbm.at[pl.ds(st, CH)], didx)
            pltpu.sync_copy(ones_v, shared.at[didx], add=True)
            return carry

        lax.fori_loop(0, NCHUNK, body, 0)
        plsc.subcore_barrier()
        pltpu.sync_copy(shared.at[pl.ds(s * OROWS, OROWS)],
                        out_hbm.at[c, pl.ds(s * OROWS, OROWS)])

        @pl.when(s == 0)
        def _():
            pltpu.sync_copy(shared.at[pl.ds(OROWS * NS, OREM)],
                            out_hbm.at[c, pl.ds(OROWS * NS, OREM)])

    return k(dst_pad, ones128, zeros128)


def _sc_agg(hs_pad, src_pad, dst_pad, zeros128):
    @functools.partial(
        pl.kernel,
        mesh=_mesh(),
        out_type=jax.ShapeDtypeStruct((NC, N, H), jnp.float32),
        scratch_types=[
            pltpu.VMEM((CH,), jnp.int32),
            pltpu.VMEM((CH,), jnp.int32),
            pltpu.VMEM((CH, H), jnp.float32),
            pltpu.VMEM_SHARED((NPAD, H), jnp.float32),
            pltpu.SemaphoreType.DMA,
        ],
    )
    def k(hs_hbm, src_hbm, dst_hbm, zeros_hbm, out_hbm,
          sidx, didx, rows, shared, sem):
        c = lax.axis_index("c")
        s = lax.axis_index("s")
        wid = c * NS + s
        pltpu.sync_copy(zeros_hbm.at[pl.ds(s * ZROWS, ZROWS)],
                        shared.at[pl.ds(s * ZROWS, ZROWS)])
        base = pl.multiple_of(wid * EPW, CH)
        plsc.subcore_barrier()

        def body(j, carry):
            st = pl.multiple_of(base + j * CH, CH)
            pltpu.sync_copy(src_hbm.at[pl.ds(st, CH)], sidx)
            pltpu.sync_copy(dst_hbm.at[pl.ds(st, CH)], didx)
            pltpu.async_copy(hs_hbm.at[sidx], rows, sem).wait()
            pltpu.sync_copy(rows, shared.at[didx], add=True)
            return carry

        lax.fori_loop(0, NCHUNK, body, 0)
        plsc.subcore_barrier()
        pltpu.sync_copy(shared.at[pl.ds(s * OROWS, OROWS)],
                        out_hbm.at[c, pl.ds(s * OROWS, OROWS)])

        @pl.when(s == 0)
        def _():
            pltpu.sync_copy(shared.at[pl.ds(OROWS * NS, OREM)],
                            out_hbm.at[c, pl.ds(OROWS * NS, OREM)])

    return k(hs_pad, src_pad, dst_pad, zeros128)


def _dinv(degs_ref):
    deg = jnp.max(degs_ref[0] + degs_ref[1], axis=1, keepdims=True) + 1.0
    return lax.rsqrt(deg)


def _tc_pre(x, W0, degs):
    def body(x_ref, w_ref, degs_ref, out_ref):
        dinv = _dinv(degs_ref)
        hs = jnp.dot(x_ref[...], w_ref[...],
                     preferred_element_type=jnp.float32) * dinv
        out_ref[pl.ds(0, N), :] = hs
        out_ref[pl.ds(N, 8), :] = jnp.zeros((8, H), jnp.float32)

    return pl.pallas_call(
        body, out_shape=jax.ShapeDtypeStruct((NP8, H), jnp.float32),
    )(x, W0, degs)


def _bn_relu(z, g, bt):
    m = jnp.mean(z, axis=0, keepdims=True)
    v = jnp.mean((z - m) ** 2, axis=0, keepdims=True)
    return jnp.maximum((z - m) * lax.rsqrt(v + EPS) * g + bt, 0.0)


def _tc_mid(agg, hs_pad, degs, b, g, bt, Wn):
    def body(agg_ref, hs_ref, degs_ref, b_ref, g_ref, bt_ref, w_ref, out_ref):
        dinv = _dinv(degs_ref)
        hs = hs_ref[pl.ds(0, N), :]
        z = (agg_ref[0] + agg_ref[1] + hs) * dinv + b_ref[...]
        h = _bn_relu(z, g_ref[...], bt_ref[...])
        out_ref[pl.ds(0, N), :] = jnp.dot(
            h, w_ref[...], preferred_element_type=jnp.float32) * dinv
        out_ref[pl.ds(N, 8), :] = jnp.zeros((8, H), jnp.float32)

    return pl.pallas_call(
        body, out_shape=jax.ShapeDtypeStruct((NP8, H), jnp.float32),
    )(agg, hs_pad, degs, b, g, bt, Wn)


def _tc_post(agg, hs_pad, degs, b, g, bt, batch8, Wo1, bo1, Wo2, bo2):
    def body(agg_ref, hs_ref, degs_ref, b_ref, g_ref, bt_ref, batch_ref,
             wo1_ref, bo1_ref, wo2_ref, bo2_ref, out_ref):
        dinv = _dinv(degs_ref)
        hs = hs_ref[pl.ds(0, N), :]
        z = (agg_ref[0] + agg_ref[1] + hs) * dinv + b_ref[...]
        h = _bn_relu(z, g_ref[...], bt_ref[...])
        bt2d = batch_ref[0:1, :]
        M = (lax.broadcasted_iota(jnp.int32, (G, N), 0) == bt2d
             ).astype(jnp.float32)
        sums = jnp.dot(M, h, preferred_element_type=jnp.float32)
        cnt = jnp.sum(M, axis=1, keepdims=True)
        pooled = sums / jnp.maximum(cnt, 1.0)
        hid = jnp.maximum(
            jnp.dot(pooled, wo1_ref[...],
                    preferred_element_type=jnp.float32) + bo1_ref[...], 0.0)
        out_ref[...] = jnp.dot(
            hid, wo2_ref[...], preferred_element_type=jnp.float32) + bo2_ref[...]

    return pl.pallas_call(
        body, out_shape=jax.ShapeDtypeStruct((G, 1), jnp.float32),
    )(agg, hs_pad, degs, b, g, bt, batch8, Wo1, bo1, Wo2, bo2)


_USE_SC_DEG = True
_USE_SC_AGG = True


def kernel(x, edge_index, batch, W0, b0, g0, bt0, W1, b1, g1, bt1,
           W2, b2, g2, bt2, Wo1, bo1, Wo2, bo2):
    pad = E_PAD - E
    fill = jnp.full((pad,), N, jnp.int32)
    srcp = jnp.concatenate([edge_index[0], fill])
    dstp = jnp.concatenate([edge_index[1], fill])
    zeros128 = jnp.zeros((NPAD, H), jnp.float32)
    ones128 = jnp.ones((CH, H), jnp.float32)
    batch8 = jnp.broadcast_to(batch[None, :], (8, N))

    if _USE_SC_DEG:
        degs = _sc_deg(dstp, ones128, zeros128)
    else:
        d = jnp.zeros((N,), jnp.float32).at[edge_index[1]].add(1.0)
        degs = jnp.stack([jnp.broadcast_to(d[:, None], (N, H)),
                          jnp.zeros((N, H), jnp.float32)])

    if _USE_SC_AGG:
        agg_fn = _sc_agg
    else:
        def agg_fn(hs_pad, srcp_, dstp_, z_):
            a = jnp.zeros((N, H), jnp.float32).at[edge_index[1]].add(
                hs_pad[edge_index[0]])
            return jnp.stack([a, jnp.zeros((N, H), jnp.float32)])
    hs = _tc_pre(x, W0, degs)

    agg = agg_fn(hs, srcp, dstp, zeros128)
    hs = _tc_mid(agg, hs, degs, b0, g0, bt0, W1)

    agg = agg_fn(hs, srcp, dstp, zeros128)
    hs = _tc_mid(agg, hs, degs, b1, g1, bt1, W2)

    agg = agg_fn(hs, srcp, dstp, zeros128)
    return _tc_post(agg, hs, degs, b2, g2, bt2, batch8, Wo1, bo1, Wo2, bo2)


# speedup vs baseline: 1.1057x; 1.1057x over previous
import functools

import jax
import jax.numpy as jnp
from jax import lax
from jax.experimental import pallas as pl
from jax.experimental.pallas import tpu as pltpu
from jax.experimental.pallas import tpu_sc as plsc

N = 10000
E = 320000
H = 128
G = 64
EPS = 1e-5

NC = 2
NS = 16
NW = NC * NS
CH = 64
EPW = ((E // NW + 2 * CH - 1) // (2 * CH)) * (2 * CH)
NCHUNK = EPW // CH
HALF = NCHUNK // 2
NITER_H = HALF // 2
E_PAD = EPW * NW
NPAD = ((N + NS * 8 - 1) // (NS * 8)) * (NS * 8)
ZROWS = NPAD // NS
OROWS = (N // NS) // 8 * 8
OREM = N - OROWS * NS
NP8 = N + 8


def _mesh():
    return plsc.VectorSubcoreMesh(core_axis_name="c", subcore_axis_name="s",
                                  num_cores=NC, num_subcores=NS)


def _sc_deg(dst_pad, ones128, zeros128):
    @functools.partial(
        pl.kernel,
        mesh=_mesh(),
        out_type=jax.ShapeDtypeStruct((NC, N, H), jnp.float32),
        scratch_types=[
            pltpu.VMEM((CH,), jnp.int32),
            pltpu.VMEM((CH, H), jnp.float32),
            pltpu.VMEM_SHARED((NPAD, H), jnp.float32),
            pltpu.SemaphoreType.DMA,
        ],
    )
    def k(dst_hbm, ones_hbm, zeros_hbm, out_hbm, didx, ones_v, shared, sem):
        c = lax.axis_index("c")
        s = lax.axis_index("s")
        wid = c * NS + s
        pltpu.sync_copy(zeros_hbm.at[pl.ds(s * ZROWS, ZROWS)],
                        shared.at[pl.ds(s * ZROWS, ZROWS)])
        pltpu.sync_copy(ones_hbm, ones_v)
        base = pl.multiple_of(wid * EPW, CH)
        plsc.subcore_barrier()

        def body(j, carry):
            st = pl.multiple_of(base + j * CH, CH)
            pltpu.sync_copy(dst_hbm.at[pl.ds(st, CH)], didx)
            pltpu.sync_copy(ones_v, shared.at[didx], add=True)
            return carry

        lax.fori_loop(0, NCHUNK, body, 0)
        plsc.subcore_barrier()
        pltpu.sync_copy(shared.at[pl.ds(s * OROWS, OROWS)],
                        out_hbm.at[c, pl.ds(s * OROWS, OROWS)])

        @pl.when(s == 0)
        def _():
            pltpu.sync_copy(shared.at[pl.ds(OROWS * NS, OREM)],
                            out_hbm.at[c, pl.ds(OROWS * NS, OREM)])

    return k(dst_pad, ones128, zeros128)


def _sc_agg(hs_pad, src_pad, dst_pad, zeros128):
    @functools.partial(
        pl.kernel,
        mesh=_mesh(),
        out_type=jax.ShapeDtypeStruct((NC, N, H), jnp.float32),
        scratch_types=[
            pltpu.VMEM((CH,), jnp.int32),
            pltpu.VMEM((CH,), jnp.int32),
            pltpu.VMEM((CH, H), jnp.float32),
            pltpu.VMEM_SHARED((NPAD, H), jnp.float32),
            pltpu.SemaphoreType.DMA,
        ],
    )
    def k(hs_hbm, src_hbm, dst_hbm, zeros_hbm, out_hbm,
          sidx, didx, rows, shared, sem):
        c = lax.axis_index("c")
        s = lax.axis_index("s")
        wid = c * NS + s
        pltpu.sync_copy(zeros_hbm.at[pl.ds(s * ZROWS, ZROWS)],
                        shared.at[pl.ds(s * ZROWS, ZROWS)])
        base = pl.multiple_of(wid * EPW, CH)
        plsc.subcore_barrier()

        def body(j, carry):
            st = pl.multiple_of(base + j * CH, CH)
            pltpu.sync_copy(src_hbm.at[pl.ds(st, CH)], sidx)
            pltpu.sync_copy(dst_hbm.at[pl.ds(st, CH)], didx)
            pltpu.async_copy(hs_hbm.at[sidx], rows, sem).wait()
            pltpu.sync_copy(rows, shared.at[didx], add=True)
            return carry

        lax.fori_loop(0, NCHUNK, body, 0)
        plsc.subcore_barrier()
        pltpu.sync_copy(shared.at[pl.ds(s * OROWS, OROWS)],
                        out_hbm.at[c, pl.ds(s * OROWS, OROWS)])

        @pl.when(s == 0)
        def _():
            pltpu.sync_copy(shared.at[pl.ds(OROWS * NS, OREM)],
                            out_hbm.at[c, pl.ds(OROWS * NS, OREM)])

    return k(hs_pad, src_pad, dst_pad, zeros128)


def _dinv(degs_ref):
    deg = jnp.max(degs_ref[0] + degs_ref[1], axis=1, keepdims=True) + 1.0
    return lax.rsqrt(deg)


def _tc_pre(x, W0, degs):
    def body(x_ref, w_ref, degs_ref, out_ref):
        dinv = _dinv(degs_ref)
        hs = jnp.dot(x_ref[...], w_ref[...],
                     preferred_element_type=jnp.float32) * dinv
        out_ref[pl.ds(0, N), :] = hs
        out_ref[pl.ds(N, 8), :] = jnp.zeros((8, H), jnp.float32)

    return pl.pallas_call(
        body, out_shape=jax.ShapeDtypeStruct((NP8, H), jnp.float32),
    )(x, W0, degs)


def _bn_relu(z, g, bt):
    m = jnp.mean(z, axis=0, keepdims=True)
    v = jnp.mean((z - m) ** 2, axis=0, keepdims=True)
    return jnp.maximum((z - m) * lax.rsqrt(v + EPS) * g + bt, 0.0)


def _tc_mid(agg, hs_pad, degs, b, g, bt, Wn):
    def body(agg_ref, hs_ref, degs_ref, b_ref, g_ref, bt_ref, w_ref, out_ref):
        dinv = _dinv(degs_ref)
        hs = hs_ref[pl.ds(0, N), :]
        z = (agg_ref[0] + agg_ref[1] + hs) * dinv + b_ref[...]
        h = _bn_relu(z, g_ref[...], bt_ref[...])
        out_ref[pl.ds(0, N), :] = jnp.dot(
            h, w_ref[...], preferred_element_type=jnp.float32) * dinv
        out_ref[pl.ds(N, 8), :] = jnp.zeros((8, H), jnp.float32)

    return pl.pallas_call(
        body, out_shape=jax.ShapeDtypeStruct((NP8, H), jnp.float32),
    )(agg, hs_pad, degs, b, g, bt, Wn)


def _tc_post(agg, hs_pad, degs, b, g, bt, batch8, Wo1, bo1, Wo2, bo2):
    def body(agg_ref, hs_ref, degs_ref, b_ref, g_ref, bt_ref, batch_ref,
             wo1_ref, bo1_ref, wo2_ref, bo2_ref, out_ref):
        dinv = _dinv(degs_ref)
        hs = hs_ref[pl.ds(0, N), :]
        z = (agg_ref[0] + agg_ref[1] + hs) * dinv + b_ref[...]
        h = _bn_relu(z, g_ref[...], bt_ref[...])
        bt2d = batch_ref[0:1, :]
        M = (lax.broadcasted_iota(jnp.int32, (G, N), 0) == bt2d
             ).astype(jnp.float32)
        sums = jnp.dot(M, h, preferred_element_type=jnp.float32)
        cnt = jnp.sum(M, axis=1, keepdims=True)
        pooled = sums / jnp.maximum(cnt, 1.0)
        hid = jnp.maximum(
            jnp.dot(pooled, wo1_ref[...],
                    preferred_element_type=jnp.float32) + bo1_ref[...], 0.0)
        out_ref[...] = jnp.dot(
            hid, wo2_ref[...], preferred_element_type=jnp.float32) + bo2_ref[...]

    return pl.pallas_call(
        body, out_shape=jax.ShapeDtypeStruct((G, 1), jnp.float32),
    )(agg, hs_pad, degs, b, g, bt, batch8, Wo1, bo1, Wo2, bo2)


_USE_SC_DEG = True
_USE_SC_AGG = True


def kernel(x, edge_index, batch, W0, b0, g0, bt0, W1, b1, g1, bt1,
           W2, b2, g2, bt2, Wo1, bo1, Wo2, bo2):
    pad = E_PAD - E
    fill = jnp.full((pad,), N, jnp.int32)
    srcp = jnp.concatenate([edge_index[0], fill])
    dstp = jnp.concatenate([edge_index[1], fill])
    zeros128 = jnp.zeros((NPAD, H), jnp.float32)
    ones128 = jnp.ones((CH, H), jnp.float32)
    batch8 = jnp.broadcast_to(batch[None, :], (8, N))

    if _USE_SC_DEG:
        degs = _sc_deg(dstp, ones128, zeros128)
    else:
        d = jnp.zeros((N,), jnp.float32).at[edge_index[1]].add(1.0)
        degs = jnp.stack([jnp.broadcast_to(d[:, None], (N, H)),
                          jnp.zeros((N, H), jnp.float32)])

    if _USE_SC_AGG:
        agg_fn = _sc_agg
    else:
        def agg_fn(hs_pad, srcp_, dstp_, z_):
            a = jnp.zeros((N, H), jnp.float32).at[edge_index[1]].add(
                hs_pad[edge_index[0]])
            return jnp.stack([a, jnp.zeros((N, H), jnp.float32)])
    hs = _tc_pre(x, W0, degs)

    agg = agg_fn(hs, srcp, dstp, zeros128)
    hs = _tc_mid(agg, hs, degs, b0, g0, bt0, W1)

    agg = agg_fn(hs, srcp, dstp, zeros128)
    hs = _tc_mid(agg, hs, degs, b1, g1, bt1, W2)

    agg = agg_fn(hs, srcp, dstp, zeros128)
    return _tc_post(agg, hs, degs, b2, g2, bt2, batch8, Wo1, bo1, Wo2, bo2)


# speedup vs baseline: 1.3288x; 1.2017x over previous
import functools

import jax
import jax.numpy as jnp
from jax import lax
from jax.experimental import pallas as pl
from jax.experimental.pallas import tpu as pltpu
from jax.experimental.pallas import tpu_sc as plsc

N = 10000
E = 320000
H = 128
G = 64
EPS = 1e-5

NC = 2
NS = 16
NW = NC * NS
CH = 128
EPW = ((E // NW + CH - 1) // CH) * CH
NCHUNK = EPW // CH
E_PAD = EPW * NW
NPAD = ((N + NS * 8 - 1) // (NS * 8)) * (NS * 8)
ZROWS = NPAD // NS
OROWS = (N // NS) // 8 * 8
OREM = N - OROWS * NS
NP8 = N + 8


def _mesh():
    return plsc.VectorSubcoreMesh(core_axis_name="c", subcore_axis_name="s",
                                  num_cores=NC, num_subcores=NS)


def _sc_deg(dst_pad, ones128, zeros128):
    @functools.partial(
        pl.kernel,
        mesh=_mesh(),
        out_type=jax.ShapeDtypeStruct((NC, N, H), jnp.float32),
        scratch_types=[
            pltpu.VMEM((CH,), jnp.int32),
            pltpu.VMEM((CH, H), jnp.float32),
            pltpu.VMEM_SHARED((NPAD, H), jnp.float32),
            pltpu.SemaphoreType.DMA,
        ],
    )
    def k(dst_hbm, ones_hbm, zeros_hbm, out_hbm, didx, ones_v, shared, sem):
        c = lax.axis_index("c")
        s = lax.axis_index("s")
        wid = c * NS + s
        pltpu.sync_copy(zeros_hbm.at[pl.ds(s * ZROWS, ZROWS)],
                        shared.at[pl.ds(s * ZROWS, ZROWS)])
        pltpu.sync_copy(ones_hbm, ones_v)
        base = pl.multiple_of(wid * EPW, CH)
        plsc.subcore_barrier()

        def body(j, carry):
            st = pl.multiple_of(base + j * CH, CH)
            pltpu.sync_copy(dst_hbm.at[pl.ds(st, CH)], didx)
            pltpu.sync_copy(ones_v, shared.at[didx], add=True)
            return carry

        lax.fori_loop(0, NCHUNK, body, 0)
        plsc.subcore_barrier()
        pltpu.sync_copy(shared.at[pl.ds(s * OROWS, OROWS)],
                        out_hbm.at[c, pl.ds(s * OROWS, OROWS)])

        @pl.when(s == 0)
        def _():
            pltpu.sync_copy(shared.at[pl.ds(OROWS * NS, OREM)],
                            out_hbm.at[c, pl.ds(OROWS * NS, OREM)])

    return k(dst_pad, ones128, zeros128)


def _sc_agg(hs_pad, src_pad, dst_pad, zeros128):
    @functools.partial(
        pl.kernel,
        mesh=_mesh(),
        out_type=jax.ShapeDtypeStruct((NC, N, H), jnp.float32),
        scratch_types=[
            pltpu.VMEM((CH,), jnp.int32),
            pltpu.VMEM((CH,), jnp.int32),
            pltpu.VMEM((CH, H), jnp.float32),
            pltpu.VMEM_SHARED((NPAD, H), jnp.float32),
            pltpu.SemaphoreType.DMA,
        ],
    )
    def k(hs_hbm, src_hbm, dst_hbm, zeros_hbm, out_hbm,
          sidx, didx, rows, shared, sem):
        c = lax.axis_index("c")
        s = lax.axis_index("s")
        wid = c * NS + s
        pltpu.sync_copy(zeros_hbm.at[pl.ds(s * ZROWS, ZROWS)],
                        shared.at[pl.ds(s * ZROWS, ZROWS)])
        base = pl.multiple_of(wid * EPW, CH)
        plsc.subcore_barrier()

        def body(j, carry):
            st = pl.multiple_of(base + j * CH, CH)
            pltpu.sync_copy(src_hbm.at[pl.ds(st, CH)], sidx)
            pltpu.sync_copy(dst_hbm.at[pl.ds(st, CH)], didx)
            pltpu.async_copy(hs_hbm.at[sidx], rows, sem).wait()
            pltpu.sync_copy(rows, shared.at[didx], add=True)
            return carry

        lax.fori_loop(0, NCHUNK, body, 0)
        plsc.subcore_barrier()
        pltpu.sync_copy(shared.at[pl.ds(s * OROWS, OROWS)],
                        out_hbm.at[c, pl.ds(s * OROWS, OROWS)])

        @pl.when(s == 0)
        def _():
            pltpu.sync_copy(shared.at[pl.ds(OROWS * NS, OREM)],
                            out_hbm.at[c, pl.ds(OROWS * NS, OREM)])

    return k(hs_pad, src_pad, dst_pad, zeros128)


def _dinv(degs_ref):
    deg = jnp.max(degs_ref[0] + degs_ref[1], axis=1, keepdims=True) + 1.0
    return lax.rsqrt(deg)


def _tc_pre(x, W0, degs):
    def body(x_ref, w_ref, degs_ref, out_ref):
        dinv = _dinv(degs_ref)
        hs = jnp.dot(x_ref[...], w_ref[...],
                     preferred_element_type=jnp.float32) * dinv
        out_ref[pl.ds(0, N), :] = hs
        out_ref[pl.ds(N, 8), :] = jnp.zeros((8, H), jnp.float32)

    return pl.pallas_call(
        body, out_shape=jax.ShapeDtypeStruct((NP8, H), jnp.float32),
    )(x, W0, degs)


def _bn_relu(z, g, bt):
    m = jnp.mean(z, axis=0, keepdims=True)
    v = jnp.mean((z - m) ** 2, axis=0, keepdims=True)
    return jnp.maximum((z - m) * lax.rsqrt(v + EPS) * g + bt, 0.0)


def _tc_mid(agg, hs_pad, degs, b, g, bt, Wn):
    def body(agg_ref, hs_ref, degs_ref, b_ref, g_ref, bt_ref, w_ref, out_ref):
        dinv = _dinv(degs_ref)
        hs = hs_ref[pl.ds(0, N), :]
        z = (agg_ref[0] + agg_ref[1] + hs) * dinv + b_ref[...]
        h = _bn_relu(z, g_ref[...], bt_ref[...])
        out_ref[pl.ds(0, N), :] = jnp.dot(
            h, w_ref[...], preferred_element_type=jnp.float32) * dinv
        out_ref[pl.ds(N, 8), :] = jnp.zeros((8, H), jnp.float32)

    return pl.pallas_call(
        body, out_shape=jax.ShapeDtypeStruct((NP8, H), jnp.float32),
    )(agg, hs_pad, degs, b, g, bt, Wn)


def _tc_post(agg, hs_pad, degs, b, g, bt, batch8, Wo1, bo1, Wo2, bo2):
    def body(agg_ref, hs_ref, degs_ref, b_ref, g_ref, bt_ref, batch_ref,
             wo1_ref, bo1_ref, wo2_ref, bo2_ref, out_ref):
        dinv = _dinv(degs_ref)
        hs = hs_ref[pl.ds(0, N), :]
        z = (agg_ref[0] + agg_ref[1] + hs) * dinv + b_ref[...]
        h = _bn_relu(z, g_ref[...], bt_ref[...])
        bt2d = batch_ref[0:1, :]
        M = (lax.broadcasted_iota(jnp.int32, (G, N), 0) == bt2d
             ).astype(jnp.float32)
        sums = jnp.dot(M, h, preferred_element_type=jnp.float32)
        cnt = jnp.sum(M, axis=1, keepdims=True)
        pooled = sums / jnp.maximum(cnt, 1.0)
        hid = jnp.maximum(
            jnp.dot(pooled, wo1_ref[...],
                    preferred_element_type=jnp.float32) + bo1_ref[...], 0.0)
        out_ref[...] = jnp.dot(
            hid, wo2_ref[...], preferred_element_type=jnp.float32) + bo2_ref[...]

    return pl.pallas_call(
        body, out_shape=jax.ShapeDtypeStruct((G, 1), jnp.float32),
    )(agg, hs_pad, degs, b, g, bt, batch8, Wo1, bo1, Wo2, bo2)


def kernel(x, edge_index, batch, W0, b0, g0, bt0, W1, b1, g1, bt1,
           W2, b2, g2, bt2, Wo1, bo1, Wo2, bo2):
    pad = E_PAD - E
    fill = jnp.full((pad,), N, jnp.int32)
    srcp = jnp.concatenate([edge_index[0], fill])
    dstp = jnp.concatenate([edge_index[1], fill])
    zeros128 = jnp.zeros((NPAD, H), jnp.float32)
    ones128 = jnp.ones((CH, H), jnp.float32)
    batch8 = jnp.broadcast_to(batch[None, :], (8, N))

    degs = _sc_deg(dstp, ones128, zeros128)
    hs = _tc_pre(x, W0, degs)

    agg = _sc_agg(hs, srcp, dstp, zeros128)
    hs = _tc_mid(agg, hs, degs, b0, g0, bt0, W1)

    agg = _sc_agg(hs, srcp, dstp, zeros128)
    hs = _tc_mid(agg, hs, degs, b1, g1, bt1, W2)

    agg = _sc_agg(hs, srcp, dstp, zeros128)
    return _tc_post(agg, hs, degs, b2, g2, bt2, batch8, Wo1, bo1, Wo2, bo2)
